# Initial kernel scaffold; baseline (speedup 1.0000x reference)
import jax
import jax.numpy as jnp
from jax.experimental import pallas as pl


def kernel(x, W1, b1, W2, b2):
    raise NotImplementedError("write your pallas kernel here")


# speedup vs baseline: 547.2471x; 547.2471x over previous
import jax
import jax.numpy as jnp
from jax.experimental import pallas as pl


def _tabgnn_kernel(x_ref, w1_ref, b1_ref, w2_ref, b2_ref, o_ref):
    x = x_ref[...]
    m = jnp.mean(x, axis=1)
    h = jnp.dot(m, w1_ref[...], preferred_element_type=jnp.float32)
    h = jnp.maximum(h + b1_ref[...], 0.0)
    o = jnp.dot(h, w2_ref[...], preferred_element_type=jnp.float32)
    o = o + b2_ref[...]
    o_ref[...] = jnp.broadcast_to(o[:, None, :], x.shape)


@jax.jit
def kernel(x, W1, b1, W2, b2):
    B, C, F = x.shape
    HID = W1.shape[1]
    BB = 64
    grid = (B // BB,)
    return pl.pallas_call(
        _tabgnn_kernel,
        grid=grid,
        in_specs=[
            pl.BlockSpec((BB, C, F), lambda i: (i, 0, 0)),
            pl.BlockSpec((F, HID), lambda i: (0, 0)),
            pl.BlockSpec((1, HID), lambda i: (0, 0)),
            pl.BlockSpec((HID, F), lambda i: (0, 0)),
            pl.BlockSpec((1, F), lambda i: (0, 0)),
        ],
        out_specs=pl.BlockSpec((BB, C, F), lambda i: (i, 0, 0)),
        out_shape=jax.ShapeDtypeStruct((B, C, F), x.dtype),
    )(x, W1, b1.reshape(1, HID), W2, b2.reshape(1, F))


# speedup vs baseline: 697.9595x; 1.2754x over previous
import jax
import jax.numpy as jnp
from jax.experimental import pallas as pl


def _tabgnn_kernel(x_ref, w1_ref, b1_ref, w2_ref, b2_ref, o_ref):
    x = x_ref[...]
    m = jnp.mean(x, axis=1)
    h = jnp.dot(m, w1_ref[...], preferred_element_type=jnp.float32)
    h = jnp.maximum(h + b1_ref[...], 0.0)
    o = jnp.dot(h, w2_ref[...], preferred_element_type=jnp.float32)
    o = o + b2_ref[...]
    o_ref[...] = jnp.broadcast_to(o[:, None, :], x.shape)


@jax.jit
def kernel(x, W1, b1, W2, b2):
    B, C, F = x.shape
    HID = W1.shape[1]
    BB = 128
    grid = (B // BB,)
    return pl.pallas_call(
        _tabgnn_kernel,
        grid=grid,
        in_specs=[
            pl.BlockSpec((BB, C, F), lambda i: (i, 0, 0)),
            pl.BlockSpec((F, HID), lambda i: (0, 0)),
            pl.BlockSpec((1, HID), lambda i: (0, 0)),
            pl.BlockSpec((HID, F), lambda i: (0, 0)),
            pl.BlockSpec((1, F), lambda i: (0, 0)),
        ],
        out_specs=pl.BlockSpec((BB, C, F), lambda i: (i, 0, 0)),
        out_shape=jax.ShapeDtypeStruct((B, C, F), x.dtype),
    )(x, W1, b1.reshape(1, HID), W2, b2.reshape(1, F))


# speedup vs baseline: 816.1870x; 1.1694x over previous
import jax
import jax.numpy as jnp
from jax.experimental import pallas as pl


def _tabgnn_kernel(x_ref, w1_ref, b1_ref, w2_ref, b2_ref, o_ref):
    x = x_ref[...]
    m = jnp.mean(x, axis=1)
    h = jnp.dot(m, w1_ref[...], preferred_element_type=jnp.float32)
    h = jnp.maximum(h + b1_ref[...], 0.0)
    o = jnp.dot(h, w2_ref[...], preferred_element_type=jnp.float32)
    o = o + b2_ref[...]
    o_ref[...] = jnp.broadcast_to(o[:, None, :], x.shape)


@jax.jit
def kernel(x, W1, b1, W2, b2):
    B, C, F = x.shape
    HID = W1.shape[1]
    BB = 256
    grid = (B // BB,)
    return pl.pallas_call(
        _tabgnn_kernel,
        grid=grid,
        in_specs=[
            pl.BlockSpec((BB, C, F), lambda i: (i, 0, 0)),
            pl.BlockSpec((F, HID), lambda i: (0, 0)),
            pl.BlockSpec((1, HID), lambda i: (0, 0)),
            pl.BlockSpec((HID, F), lambda i: (0, 0)),
            pl.BlockSpec((1, F), lambda i: (0, 0)),
        ],
        out_specs=pl.BlockSpec((BB, C, F), lambda i: (i, 0, 0)),
        out_shape=jax.ShapeDtypeStruct((B, C, F), x.dtype),
    )(x, W1, b1.reshape(1, HID), W2, b2.reshape(1, F))
